# baseline (speedup 1.0000x reference)
import jax
import jax.numpy as jnp
from jax import lax
from jax.experimental import pallas as pl
from jax.experimental.pallas import tpu as pltpu
from jax.experimental.pallas import tpu_sc as plsc

NUM_CORES = 2
NUM_WORKERS = 32
HIST_LEN = 200
EMBED_DIM = 32
LANES = 128
OUT_LEN = (1 + HIST_LEN) * EMBED_DIM
HI_ROWS = 7
LO_ROWS = 6
MAX_ROWS = HI_ROWS


def _worker_base(w):
    return w * HI_ROWS if w < 8 else 8 * HI_ROWS + (w - 8) * LO_ROWS


def _worker_rows(w):
    return HI_ROWS if w < 8 else LO_ROWS


def _extract_column(block, col, rowbuf, offset):
    for h in range(EMBED_DIM // 16):
        idx_d = lax.iota(jnp.int32, 16) + (h * 16)
        idx_c = jnp.zeros((16,), jnp.int32) + col
        vals = plsc.load_gather(block, [idx_d, idx_c])
        rowbuf[pl.ds(offset + h * 16, 16)] = vals


def _start_idx_fetch(movie_idx, idx_v, base, n_rows, sem):
    abase = (base // 8) * 8
    sh = base - abase
    cp = pltpu.async_copy(movie_idx.at[pl.ds(abase, sh + n_rows)],
                          idx_v.at[pl.ds(0, sh + n_rows)], sem)
    return cp, sh


def _gather_rows(iv, sh, n_rows, base, tabT, out, blocks, rowbuf, sem,
                 extra_copies=()):
    copies = []
    for j in range(n_rows):
        i = iv[sh + j]
        t = pl.multiple_of((i // LANES) * LANES, LANES)
        copies.append(pltpu.async_copy(
            tabT.at[:, pl.ds(t, LANES)], blocks.at[j], sem))
    for c in extra_copies:
        c.wait()
    for c in copies:
        c.wait()
    for j in range(n_rows):
        _extract_column(blocks.at[j], iv[sh + j] % LANES, rowbuf,
                        j * EMBED_DIM)
    pltpu.sync_copy(
        rowbuf.at[pl.ds(0, n_rows * EMBED_DIM)],
        out.at[pl.ds(EMBED_DIM + base * EMBED_DIM, n_rows * EMBED_DIM)])


def _gather_body(user_idx, movie_idx, user_tabT, movie_tabT, out,
                 idx_v, uidx_v, blocks, ublock, rowbuf, urowbuf, sem, usem):
    wid = lax.axis_index("s") * NUM_CORES + lax.axis_index("c")

    def _make_branch(w):
        n, base = _worker_rows(w), _worker_base(w)

        @pl.when(wid == w)
        def _():
            cp, sh = _start_idx_fetch(movie_idx, idx_v, base, n, sem)
            cp.wait()
            iv = idx_v[pl.ds(0, 16)]
            _gather_rows(iv, sh, n, base, movie_tabT, out, blocks,
                         rowbuf, sem)

    for w in range(NUM_WORKERS - 1):
        _make_branch(w)

    @pl.when(wid == NUM_WORKERS - 1)
    def _():
        base = _worker_base(NUM_WORKERS - 1)
        ucp = pltpu.async_copy(user_idx, uidx_v.at[pl.ds(0, 1)], usem)
        mcp, sh = _start_idx_fetch(movie_idx, idx_v, base, LO_ROWS, sem)
        ucp.wait()
        mcp.wait()
        iv = idx_v[pl.ds(0, 16)]
        ui = uidx_v[...][0]
        ut = pl.multiple_of((ui // LANES) * LANES, LANES)
        ucopy = pltpu.async_copy(
            user_tabT.at[:, pl.ds(ut, LANES)], ublock, usem)
        _gather_rows(iv, sh, LO_ROWS, base, movie_tabT, out, blocks,
                     rowbuf, sem, extra_copies=(ucopy,))
        _extract_column(ublock, ui % LANES, urowbuf, 0)
        pltpu.sync_copy(urowbuf, out.at[pl.ds(0, EMBED_DIM)])


@jax.jit
def kernel(user, movie_history, user_table, movie_table):
    mesh = plsc.VectorSubcoreMesh(core_axis_name="c", subcore_axis_name="s")
    flat = pl.kernel(
        _gather_body,
        out_type=jax.ShapeDtypeStruct((OUT_LEN,), jnp.float32),
        mesh=mesh,
        scratch_types=[
            pltpu.VMEM((16,), jnp.int32),
            pltpu.VMEM((16,), jnp.int32),
            pltpu.VMEM((MAX_ROWS, EMBED_DIM, LANES), jnp.float32),
            pltpu.VMEM((EMBED_DIM, LANES), jnp.float32),
            pltpu.VMEM((MAX_ROWS * EMBED_DIM,), jnp.float32),
            pltpu.VMEM((EMBED_DIM,), jnp.float32),
            pltpu.SemaphoreType.DMA,
            pltpu.SemaphoreType.DMA,
        ],
        compiler_params=pltpu.CompilerParams(needs_layout_passes=False),
    )(user, movie_history, user_table.T, movie_table.T)
    return flat.reshape(1, OUT_LEN)


# speedup vs baseline: 1.3866x; 1.3866x over previous
import jax
import jax.numpy as jnp
from jax import lax
from jax.experimental import pallas as pl
from jax.experimental.pallas import tpu as pltpu
from jax.experimental.pallas import tpu_sc as plsc

NUM_CORES = 2
HIST_LEN = 200
ROWS_PER_WORKER = 8
NUM_MOVIE_WORKERS = HIST_LEN // ROWS_PER_WORKER
EMBED_DIM = 32
LANES = 128
OUT_LEN = (1 + HIST_LEN) * EMBED_DIM


def _extract_column(block, col, rowbuf, offset):
    for h in range(EMBED_DIM // 16):
        idx_d = lax.iota(jnp.int32, 16) + (h * 16)
        idx_c = jnp.zeros((16,), jnp.int32) + col
        vals = plsc.load_gather(block, [idx_d, idx_c])
        rowbuf[pl.ds(offset + h * 16, 16)] = vals


def _gather_body(user_idx, movie_idx, user_tabT, movie_tabT, out,
                 idx_v, blocks, rowbuf, sem):
    wid = lax.axis_index("s") * NUM_CORES + lax.axis_index("c")

    @pl.when(wid < NUM_MOVIE_WORKERS)
    def _():
        base = wid * ROWS_PER_WORKER
        pltpu.sync_copy(movie_idx.at[pl.ds(base, ROWS_PER_WORKER)],
                        idx_v.at[pl.ds(0, ROWS_PER_WORKER)])
        iv = idx_v[...]
        copies = []
        for j in range(ROWS_PER_WORKER):
            i = iv[j]
            t = pl.multiple_of((i // LANES) * LANES, LANES)
            copies.append(pltpu.async_copy(
                movie_tabT.at[:, pl.ds(t, LANES)], blocks.at[j], sem))
        for c in copies:
            c.wait()
        for j in range(ROWS_PER_WORKER):
            col = iv[j] % LANES
            _extract_column(blocks.at[j], col, rowbuf, j * EMBED_DIM)
        pltpu.sync_copy(
            rowbuf,
            out.at[pl.ds(EMBED_DIM + base * EMBED_DIM,
                         ROWS_PER_WORKER * EMBED_DIM)])

    @pl.when(wid == NUM_MOVIE_WORKERS)
    def _():
        pltpu.sync_copy(user_idx, idx_v.at[pl.ds(0, 1)])
        i = idx_v[...][0]
        t = pl.multiple_of((i // LANES) * LANES, LANES)
        pltpu.async_copy(
            user_tabT.at[:, pl.ds(t, LANES)], blocks.at[0], sem).wait()
        _extract_column(blocks.at[0], i % LANES, rowbuf, 0)
        pltpu.sync_copy(rowbuf.at[pl.ds(0, EMBED_DIM)],
                        out.at[pl.ds(0, EMBED_DIM)])


@jax.jit
def kernel(user, movie_history, user_table, movie_table):
    mesh = plsc.VectorSubcoreMesh(core_axis_name="c", subcore_axis_name="s")
    flat = pl.kernel(
        _gather_body,
        out_type=jax.ShapeDtypeStruct((OUT_LEN,), jnp.float32),
        mesh=mesh,
        scratch_types=[
            pltpu.VMEM((16,), jnp.int32),
            pltpu.VMEM((ROWS_PER_WORKER, EMBED_DIM, LANES), jnp.float32),
            pltpu.VMEM((ROWS_PER_WORKER * EMBED_DIM,), jnp.float32),
            pltpu.SemaphoreType.DMA,
        ],
        compiler_params=pltpu.CompilerParams(needs_layout_passes=False),
    )(user, movie_history, user_table.T, movie_table.T)
    return flat.reshape(1, OUT_LEN)


# speedup vs baseline: 1.3891x; 1.0018x over previous
import jax
import jax.numpy as jnp
from jax import lax
from jax.experimental import pallas as pl
from jax.experimental.pallas import tpu as pltpu
from jax.experimental.pallas import tpu_sc as plsc

NUM_CORES = 2
HIST_LEN = 200
ROWS_PER_WORKER = 8
NUM_MOVIE_WORKERS = HIST_LEN // ROWS_PER_WORKER
EMBED_DIM = 32
LANES = 128
OUT_LEN = (1 + HIST_LEN) * EMBED_DIM


def _extract_column(block, col, rowbuf, offset):
    for h in range(EMBED_DIM // 16):
        idx_d = lax.iota(jnp.int32, 16) + (h * 16)
        idx_c = jnp.zeros((16,), jnp.int32) + col
        vals = plsc.load_gather(block, [idx_d, idx_c])
        rowbuf[pl.ds(offset + h * 16, 16)] = vals


def _gather_body(user_idx, movie_idx, user_tabT, movie_tabT, out,
                 idx_v, blocks, rowbuf, sem):
    wid = lax.axis_index("s") * NUM_CORES + lax.axis_index("c")

    @pl.when(wid < NUM_MOVIE_WORKERS)
    def _():
        base = wid * ROWS_PER_WORKER
        pltpu.sync_copy(movie_idx.at[pl.ds(base, ROWS_PER_WORKER)],
                        idx_v.at[pl.ds(0, ROWS_PER_WORKER)])
        iv = idx_v[...]
        copies = []
        for j in range(ROWS_PER_WORKER):
            i = iv[j]
            t = pl.multiple_of((i // LANES) * LANES, LANES)
            copies.append(pltpu.async_copy(
                movie_tabT.at[:, pl.ds(t, LANES)], blocks.at[j], sem))
        for c in copies:
            c.wait()
        for j in range(ROWS_PER_WORKER):
            col = iv[j] % LANES
            _extract_column(blocks.at[j], col, rowbuf, j * EMBED_DIM)
        pltpu.sync_copy(
            rowbuf,
            out.at[pl.ds(EMBED_DIM + base * EMBED_DIM,
                         ROWS_PER_WORKER * EMBED_DIM)])

    @pl.when(wid == NUM_MOVIE_WORKERS)
    def _():
        pltpu.sync_copy(user_idx, idx_v.at[pl.ds(0, 1)])
        i = idx_v[...][0]
        t = pl.multiple_of((i // LANES) * LANES, LANES)
        pltpu.async_copy(
            user_tabT.at[:, pl.ds(t, LANES)], blocks.at[0], sem).wait()
        _extract_column(blocks.at[0], i % LANES, rowbuf, 0)
        pltpu.sync_copy(rowbuf.at[pl.ds(0, EMBED_DIM)],
                        out.at[pl.ds(0, EMBED_DIM)])


@jax.jit
def kernel(user, movie_history, user_table, movie_table):
    mesh = plsc.VectorSubcoreMesh(core_axis_name="c", subcore_axis_name="s")
    flat = pl.kernel(
        _gather_body,
        out_type=jax.ShapeDtypeStruct((OUT_LEN,), jnp.float32),
        mesh=mesh,
        scratch_types=[
            pltpu.VMEM((16,), jnp.int32),
            pltpu.VMEM((ROWS_PER_WORKER, EMBED_DIM, LANES), jnp.float32),
            pltpu.VMEM((ROWS_PER_WORKER * EMBED_DIM,), jnp.float32),
            pltpu.SemaphoreType.DMA,
        ],
        compiler_params=pltpu.CompilerParams(
            needs_layout_passes=False,
            disable_bounds_checks=True,
        ),
    )(user, movie_history, user_table.T, movie_table.T)
    return flat.reshape(1, OUT_LEN)
